# baseline (speedup 1.0000x reference)
import functools

import numpy as np
import jax
import jax.numpy as jnp
from jax import lax
from jax.experimental import pallas as pl
from jax.experimental.pallas import tpu as pltpu
from jax.experimental.pallas import tpu_sc as plsc

_TOP_K = 32
_MASK_RATIO = 0.5
_NEG = -3.0e38


def _rotl32(x, d):
    return ((x << np.uint32(d)) | (x >> np.uint32(32 - d))).astype(np.uint32)


def _threefry2x32_np(k0, k1, x0, x1):
    rot = [[13, 15, 26, 6], [17, 29, 16, 24]]
    ks2 = np.uint32(k0 ^ k1 ^ np.uint32(0x1BD11BDA))
    x0 = (x0 + k0).astype(np.uint32)
    x1 = (x1 + k1).astype(np.uint32)
    sched = [(k1, ks2), (ks2, k0), (k0, k1), (k1, ks2), (ks2, k0)]
    for g in range(5):
        for r in rot[g % 2]:
            x0 = (x0 + x1).astype(np.uint32)
            x1 = _rotl32(x1, r)
            x1 = x1 ^ x0
        a, b = sched[g]
        x0 = (x0 + a).astype(np.uint32)
        x1 = (x1 + b + np.uint32(g + 1)).astype(np.uint32)
    return x0, x1


@functools.lru_cache(maxsize=2)
def _bernoulli_mask_np(n: int):
    size = n * n
    cnt = np.arange(size, dtype=np.uint64)
    hi = (cnt >> np.uint64(32)).astype(np.uint32)
    lo = cnt.astype(np.uint32)
    b0, b1 = _threefry2x32_np(np.uint32(0), np.uint32(42), hi, lo)
    bits = b0 ^ b1
    f = ((bits >> np.uint32(9)) | np.uint32(0x3F800000)).view(np.float32) - np.float32(1.0)
    return (f < _MASK_RATIO).astype(np.int8).reshape(n, n)


def _sc_scatter_adj(r_arr, c_arr, n):
    e = r_arr.shape[0]
    info = plsc.get_sparse_core_info()
    ncores = info.num_cores
    nw = ncores * info.num_subcores
    rpw = n // nw
    sub_rows = min(8, rpw)
    nsub = rpw // sub_rows
    slab_w = sub_rows * n
    chunk = min(2048, e)
    nchunk = e // chunk
    nvec_chunk = chunk // 16
    cap = e + 16

    mesh = plsc.VectorSubcoreMesh(core_axis_name="c", subcore_axis_name="s")

    @functools.partial(
        pl.kernel, mesh=mesh,
        compiler_params=pltpu.CompilerParams(needs_layout_passes=False),
        out_type=jax.ShapeDtypeStruct((n * n,), jnp.float32),
        scratch_types=[
            pltpu.VMEM((chunk,), jnp.int32),
            pltpu.VMEM((chunk,), jnp.int32),
            pltpu.VMEM((cap,), jnp.int32),
            pltpu.VMEM((slab_w,), jnp.float32),
        ],
    )
    def sck(r_hbm, c_hbm, a_hbm, r_vm, c_vm, ci_vm, slab_vm):
        wid = lax.axis_index("s") * ncores + lax.axis_index("c")
        base = wid * rpw
        zeros16 = jnp.zeros((16,), jnp.float32)
        ones16 = jnp.full((16,), 1.0, jnp.float32)
        sent16 = jnp.full((16,), -(2 ** 24), jnp.int32)

        def chunk_body(ch, off):
            pltpu.sync_copy(r_hbm.at[pl.ds(ch * chunk, chunk)], r_vm)
            pltpu.sync_copy(c_hbm.at[pl.ds(ch * chunk, chunk)], c_vm)

            def vec_body(j, off):
                r16 = r_vm[pl.ds(j * 16, 16)]
                c16 = c_vm[pl.ds(j * 16, 16)]
                lr = r16 - base
                m = jnp.logical_and(lr >= 0, lr < rpw)
                idx = lr * n + c16
                mi = m.astype(jnp.int32)
                pos = plsc.cumsum(mi)
                plsc.store_scatter(ci_vm, [off + pos - 1], idx, mask=m)
                return off + jnp.sum(mi)

            return lax.fori_loop(0, nvec_chunk, vec_body, off)

        off = lax.fori_loop(0, nchunk, chunk_body, jnp.int32(0))
        ci_vm[pl.ds(off, 16)] = sent16
        nvec = (off + 15) // 16

        def sub_body(s, _):
            def zbody(t, _):
                b = t * 128
                for u in range(8):
                    slab_vm[pl.ds(b + u * 16, 16)] = zeros16
                return 0

            lax.fori_loop(0, slab_w // 128, zbody, 0)
            lo = s * slab_w

            def sbody(j, _):
                idx = ci_vm[pl.ds(j * 16, 16)]
                loc = idx - lo
                m = jnp.logical_and(loc >= 0, loc < slab_w)
                plsc.addupdate_scatter(
                    slab_vm, [jnp.where(m, loc, 0)], ones16, mask=m)
                return 0

            lax.fori_loop(0, nvec, sbody, 0)
            pltpu.sync_copy(
                slab_vm,
                a_hbm.at[pl.ds((base + s * sub_rows) * n, slab_w)])
            return 0

        lax.fori_loop(0, nsub, sub_body, 0)

    return sck(r_arr, c_arr)


def _mask_body(x_ref, bern_ref, out_ref, xn_s, *, rb):
    i = pl.program_id(0)
    n = x_ref.shape[0]

    @pl.when(i == 0)
    def _():
        xv = x_ref[...]
        nrm = jnp.sqrt(jnp.sum(xv * xv, axis=1, keepdims=True))
        xn_s[...] = xv / jnp.maximum(nrm, 1e-8)

    xb = xn_s[pl.ds(i * rb, rb), :]
    sim = jax.lax.dot_general(
        xb, xn_s[...], (((1,), (1,)), ((), ())),
        preferred_element_type=jnp.float32)
    th = jnp.max(sim, axis=1, keepdims=True)
    for _ in range(_TOP_K - 1):
        th = jnp.max(jnp.where(sim < th, sim, _NEG), axis=1, keepdims=True)
    mask = jnp.logical_or(bern_ref[...].astype(jnp.int32) > 0, sim >= th)
    out_ref[...] = mask.astype(jnp.int8)


def _a2_body(a_row, a_full, out):
    out[...] = jax.lax.dot_general(
        a_row[...], a_full[...], (((1,), (0,)), ((), ())),
        preferred_element_type=jnp.float32).astype(jnp.bfloat16)


def _a3_agg_body(a2_row, a_full, x_ref, mask_ref, alpha_ref,
                 fw_ref, fb_ref, l0w_ref, l0b_ref, l1w_ref, l1b_ref,
                 out_ref, *, rb):
    i = pl.program_id(0)
    n = x_ref.shape[0]
    row0 = i * rb

    a3f = jax.lax.dot_general(
        a2_row[...], a_full[...], (((1,), (0,)), ((), ())),
        preferred_element_type=jnp.float32)

    mf = (mask_ref[...].astype(jnp.int32) > 0).astype(jnp.float32)
    counts = jnp.sum(mf, axis=1, keepdims=True)

    a2f = a2_row[...].astype(jnp.float32)
    al0 = alpha_ref[0]
    al1 = alpha_ref[1]
    al2 = alpha_ref[2]
    wgt = mf * (al1 * a2f + al2 * a3f)
    col_id = jax.lax.broadcasted_iota(jnp.int32, (rb, n), 1)
    row_id = jax.lax.broadcasted_iota(jnp.int32, (rb, n), 0)
    wgt = wgt + jnp.where(col_id == row_id + row0, al0 * mf, 0.0)

    fused = jax.lax.dot_general(
        wgt, x_ref[...], (((1,), (0,)), ((), ())),
        preferred_element_type=jnp.float32)
    fused = fused / (counts + 1e-6)

    xo = x_ref[pl.ds(row0, rb), :]
    in_ch = xo.shape[1]
    o1 = (jax.lax.dot_general(xo, fw_ref[:, :in_ch],
                              (((1,), (1,)), ((), ())),
                              preferred_element_type=jnp.float32)
          + jax.lax.dot_general(fused, fw_ref[:, in_ch:],
                                (((1,), (1,)), ((), ())),
                                preferred_element_type=jnp.float32)
          + fb_ref[...])
    o2 = jax.lax.dot_general(o1, l0w_ref[...], (((1,), (1,)), ((), ())),
                             preferred_element_type=jnp.float32)
    o2 = jnp.maximum(o2 + l0b_ref[...], 0.0)
    o3 = jax.lax.dot_general(o2, l1w_ref[...], (((1,), (1,)), ((), ())),
                             preferred_element_type=jnp.float32)
    o3 = o3 + l1b_ref[...]
    mx = jnp.max(o3, axis=1, keepdims=True)
    sh = o3 - mx
    out_ref[...] = sh - jnp.log(jnp.sum(jnp.exp(sh), axis=1, keepdims=True))


def kernel(x, edge_index, alpha, fusion_W, fusion_b, lin0_W, lin0_b,
           lin1_W, lin1_b):
    n, in_ch = x.shape
    out_ch = lin1_W.shape[0]
    rb = 256 if n % 256 == 0 else n
    nb = n // rb

    a_f32 = _sc_scatter_adj(edge_index[0], edge_index[1], n).reshape(n, n)

    bern8 = jnp.asarray(_bernoulli_mask_np(n))
    mask8 = pl.pallas_call(
        functools.partial(_mask_body, rb=rb),
        grid=(nb,),
        in_specs=[
            pl.BlockSpec((n, in_ch), lambda i: (0, 0)),
            pl.BlockSpec((rb, n), lambda i: (i, 0)),
        ],
        out_specs=pl.BlockSpec((rb, n), lambda i: (i, 0)),
        out_shape=jax.ShapeDtypeStruct((n, n), jnp.int8),
        scratch_shapes=[pltpu.VMEM((n, in_ch), jnp.float32)],
    )(x, bern8)

    a_bf = a_f32.astype(jnp.bfloat16)

    a2 = pl.pallas_call(
        _a2_body,
        grid=(nb,),
        in_specs=[
            pl.BlockSpec((rb, n), lambda i: (i, 0)),
            pl.BlockSpec((n, n), lambda i: (0, 0)),
        ],
        out_specs=pl.BlockSpec((rb, n), lambda i: (i, 0)),
        out_shape=jax.ShapeDtypeStruct((n, n), jnp.bfloat16),
    )(a_bf, a_bf)

    out = pl.pallas_call(
        functools.partial(_a3_agg_body, rb=rb),
        grid=(nb,),
        in_specs=[
            pl.BlockSpec((rb, n), lambda i: (i, 0)),
            pl.BlockSpec((n, n), lambda i: (0, 0)),
            pl.BlockSpec((n, in_ch), lambda i: (0, 0)),
            pl.BlockSpec((rb, n), lambda i: (i, 0)),
            pl.BlockSpec(memory_space=pltpu.SMEM),
            pl.BlockSpec((fusion_W.shape[0], 2 * in_ch), lambda i: (0, 0)),
            pl.BlockSpec((1, fusion_b.shape[0]), lambda i: (0, 0)),
            pl.BlockSpec(lin0_W.shape, lambda i: (0, 0)),
            pl.BlockSpec((1, lin0_b.shape[0]), lambda i: (0, 0)),
            pl.BlockSpec(lin1_W.shape, lambda i: (0, 0)),
            pl.BlockSpec((1, lin1_b.shape[0]), lambda i: (0, 0)),
        ],
        out_specs=pl.BlockSpec((rb, out_ch), lambda i: (i, 0)),
        out_shape=jax.ShapeDtypeStruct((n, out_ch), jnp.float32),
    )(a2, a_bf, x, mask8, alpha,
      fusion_W, fusion_b.reshape(1, -1),
      lin0_W, lin0_b.reshape(1, -1),
      lin1_W, lin1_b.reshape(1, -1))
    return out


# speedup vs baseline: 1.0007x; 1.0007x over previous
import functools

import numpy as np
import jax
import jax.numpy as jnp
from jax import lax
from jax.experimental import pallas as pl
from jax.experimental.pallas import tpu as pltpu
from jax.experimental.pallas import tpu_sc as plsc

_TOP_K = 32
_MASK_RATIO = 0.5
_NEG = -3.0e38


def _rotl32(x, d):
    return ((x << np.uint32(d)) | (x >> np.uint32(32 - d))).astype(np.uint32)


def _threefry2x32_np(k0, k1, x0, x1):
    rot = [[13, 15, 26, 6], [17, 29, 16, 24]]
    ks2 = np.uint32(k0 ^ k1 ^ np.uint32(0x1BD11BDA))
    x0 = (x0 + k0).astype(np.uint32)
    x1 = (x1 + k1).astype(np.uint32)
    sched = [(k1, ks2), (ks2, k0), (k0, k1), (k1, ks2), (ks2, k0)]
    for g in range(5):
        for r in rot[g % 2]:
            x0 = (x0 + x1).astype(np.uint32)
            x1 = _rotl32(x1, r)
            x1 = x1 ^ x0
        a, b = sched[g]
        x0 = (x0 + a).astype(np.uint32)
        x1 = (x1 + b + np.uint32(g + 1)).astype(np.uint32)
    return x0, x1


@functools.lru_cache(maxsize=2)
def _bernoulli_mask_np(n: int):
    size = n * n
    cnt = np.arange(size, dtype=np.uint64)
    hi = (cnt >> np.uint64(32)).astype(np.uint32)
    lo = cnt.astype(np.uint32)
    b0, b1 = _threefry2x32_np(np.uint32(0), np.uint32(42), hi, lo)
    bits = b0 ^ b1
    f = ((bits >> np.uint32(9)) | np.uint32(0x3F800000)).view(np.float32) - np.float32(1.0)
    return (f < _MASK_RATIO).astype(np.int8).reshape(n, n)


def _sc_scatter_adj(r_arr, c_arr, n):
    e = r_arr.shape[0]
    info = plsc.get_sparse_core_info()
    ncores = info.num_cores
    nw = ncores * info.num_subcores
    rpw = n // nw
    sub_rows = min(8, rpw)
    nsub = rpw // sub_rows
    slab_w = sub_rows * n
    chunk = min(8192, e)
    nchunk = e // chunk
    nvec_chunk = chunk // 16
    cap = e + 16

    mesh = plsc.VectorSubcoreMesh(core_axis_name="c", subcore_axis_name="s")

    @functools.partial(
        pl.kernel, mesh=mesh,
        compiler_params=pltpu.CompilerParams(needs_layout_passes=False),
        out_type=jax.ShapeDtypeStruct((n * n,), jnp.float32),
        scratch_types=[
            pltpu.VMEM((chunk,), jnp.int32),
            pltpu.VMEM((chunk,), jnp.int32),
            pltpu.VMEM((cap,), jnp.int32),
            pltpu.VMEM((slab_w,), jnp.float32),
        ],
    )
    def sck(r_hbm, c_hbm, a_hbm, r_vm, c_vm, ci_vm, slab_vm):
        wid = lax.axis_index("s") * ncores + lax.axis_index("c")
        base = wid * rpw
        zeros16 = jnp.zeros((16,), jnp.float32)
        ones16 = jnp.full((16,), 1.0, jnp.float32)
        sent16 = jnp.full((16,), -(2 ** 24), jnp.int32)

        def chunk_body(ch, off):
            pltpu.sync_copy(r_hbm.at[pl.ds(ch * chunk, chunk)], r_vm)
            pltpu.sync_copy(c_hbm.at[pl.ds(ch * chunk, chunk)], c_vm)

            def vec_body(j, off):
                r16 = r_vm[pl.ds(j * 16, 16)]
                c16 = c_vm[pl.ds(j * 16, 16)]
                lr = r16 - base
                m = jnp.logical_and(lr >= 0, lr < rpw)
                idx = lr * n + c16
                mi = m.astype(jnp.int32)
                pos = plsc.cumsum(mi)
                plsc.store_scatter(ci_vm, [off + pos - 1], idx, mask=m)
                return off + jnp.sum(mi)

            return lax.fori_loop(0, nvec_chunk, vec_body, off)

        off = lax.fori_loop(0, nchunk, chunk_body, jnp.int32(0))
        ci_vm[pl.ds(off, 16)] = sent16
        nvec = (off + 15) // 16

        def sub_body(s, _):
            def zbody(t, _):
                b = t * 128
                for u in range(8):
                    slab_vm[pl.ds(b + u * 16, 16)] = zeros16
                return 0

            lax.fori_loop(0, slab_w // 128, zbody, 0)
            lo = s * slab_w

            def sbody(j, _):
                idx = ci_vm[pl.ds(j * 16, 16)]
                loc = idx - lo
                m = jnp.logical_and(loc >= 0, loc < slab_w)
                plsc.addupdate_scatter(
                    slab_vm, [jnp.where(m, loc, 0)], ones16, mask=m)
                return 0

            lax.fori_loop(0, nvec, sbody, 0)
            pltpu.sync_copy(
                slab_vm,
                a_hbm.at[pl.ds((base + s * sub_rows) * n, slab_w)])
            return 0

        lax.fori_loop(0, nsub, sub_body, 0)

    return sck(r_arr, c_arr)


def _mask_body(x_ref, bern_ref, out_ref, xn_s, *, rb):
    i = pl.program_id(0)
    n = x_ref.shape[0]

    @pl.when(i == 0)
    def _():
        xv = x_ref[...]
        nrm = jnp.sqrt(jnp.sum(xv * xv, axis=1, keepdims=True))
        xn_s[...] = xv / jnp.maximum(nrm, 1e-8)

    xb = xn_s[pl.ds(i * rb, rb), :]
    sim = jax.lax.dot_general(
        xb, xn_s[...], (((1,), (1,)), ((), ())),
        preferred_element_type=jnp.float32)
    th = jnp.max(sim, axis=1, keepdims=True)
    for _ in range(_TOP_K - 1):
        th = jnp.max(jnp.where(sim < th, sim, _NEG), axis=1, keepdims=True)
    mask = jnp.logical_or(bern_ref[...].astype(jnp.int32) > 0, sim >= th)
    out_ref[...] = mask.astype(jnp.int8)


def _a2_body(a_row, a_full, out):
    out[...] = jax.lax.dot_general(
        a_row[...], a_full[...], (((1,), (0,)), ((), ())),
        preferred_element_type=jnp.float32).astype(jnp.bfloat16)


def _a3_agg_body(a2_row, a_full, x_ref, mask_ref, alpha_ref,
                 fw_ref, fb_ref, l0w_ref, l0b_ref, l1w_ref, l1b_ref,
                 out_ref, *, rb):
    i = pl.program_id(0)
    n = x_ref.shape[0]
    row0 = i * rb

    a3f = jax.lax.dot_general(
        a2_row[...], a_full[...], (((1,), (0,)), ((), ())),
        preferred_element_type=jnp.float32)

    mf = (mask_ref[...].astype(jnp.int32) > 0).astype(jnp.float32)
    counts = jnp.sum(mf, axis=1, keepdims=True)

    a2f = a2_row[...].astype(jnp.float32)
    al0 = alpha_ref[0]
    al1 = alpha_ref[1]
    al2 = alpha_ref[2]
    wgt = mf * (al1 * a2f + al2 * a3f)
    col_id = jax.lax.broadcasted_iota(jnp.int32, (rb, n), 1)
    row_id = jax.lax.broadcasted_iota(jnp.int32, (rb, n), 0)
    wgt = wgt + jnp.where(col_id == row_id + row0, al0 * mf, 0.0)

    fused = jax.lax.dot_general(
        wgt, x_ref[...], (((1,), (0,)), ((), ())),
        preferred_element_type=jnp.float32)
    fused = fused / (counts + 1e-6)

    xo = x_ref[pl.ds(row0, rb), :]
    in_ch = xo.shape[1]
    o1 = (jax.lax.dot_general(xo, fw_ref[:, :in_ch],
                              (((1,), (1,)), ((), ())),
                              preferred_element_type=jnp.float32)
          + jax.lax.dot_general(fused, fw_ref[:, in_ch:],
                                (((1,), (1,)), ((), ())),
                                preferred_element_type=jnp.float32)
          + fb_ref[...])
    o2 = jax.lax.dot_general(o1, l0w_ref[...], (((1,), (1,)), ((), ())),
                             preferred_element_type=jnp.float32)
    o2 = jnp.maximum(o2 + l0b_ref[...], 0.0)
    o3 = jax.lax.dot_general(o2, l1w_ref[...], (((1,), (1,)), ((), ())),
                             preferred_element_type=jnp.float32)
    o3 = o3 + l1b_ref[...]
    mx = jnp.max(o3, axis=1, keepdims=True)
    sh = o3 - mx
    out_ref[...] = sh - jnp.log(jnp.sum(jnp.exp(sh), axis=1, keepdims=True))


def kernel(x, edge_index, alpha, fusion_W, fusion_b, lin0_W, lin0_b,
           lin1_W, lin1_b):
    n, in_ch = x.shape
    out_ch = lin1_W.shape[0]
    rb = 256 if n % 256 == 0 else n
    nb = n // rb

    a_f32 = _sc_scatter_adj(edge_index[0], edge_index[1], n).reshape(n, n)

    bern8 = jnp.asarray(_bernoulli_mask_np(n))
    mask8 = pl.pallas_call(
        functools.partial(_mask_body, rb=rb),
        grid=(nb,),
        in_specs=[
            pl.BlockSpec((n, in_ch), lambda i: (0, 0)),
            pl.BlockSpec((rb, n), lambda i: (i, 0)),
        ],
        out_specs=pl.BlockSpec((rb, n), lambda i: (i, 0)),
        out_shape=jax.ShapeDtypeStruct((n, n), jnp.int8),
        scratch_shapes=[pltpu.VMEM((n, in_ch), jnp.float32)],
    )(x, bern8)

    a_bf = a_f32.astype(jnp.bfloat16)

    a2 = pl.pallas_call(
        _a2_body,
        grid=(nb,),
        in_specs=[
            pl.BlockSpec((rb, n), lambda i: (i, 0)),
            pl.BlockSpec((n, n), lambda i: (0, 0)),
        ],
        out_specs=pl.BlockSpec((rb, n), lambda i: (i, 0)),
        out_shape=jax.ShapeDtypeStruct((n, n), jnp.bfloat16),
    )(a_bf, a_bf)

    out = pl.pallas_call(
        functools.partial(_a3_agg_body, rb=rb),
        grid=(nb,),
        in_specs=[
            pl.BlockSpec((rb, n), lambda i: (i, 0)),
            pl.BlockSpec((n, n), lambda i: (0, 0)),
            pl.BlockSpec((n, in_ch), lambda i: (0, 0)),
            pl.BlockSpec((rb, n), lambda i: (i, 0)),
            pl.BlockSpec(memory_space=pltpu.SMEM),
            pl.BlockSpec((fusion_W.shape[0], 2 * in_ch), lambda i: (0, 0)),
            pl.BlockSpec((1, fusion_b.shape[0]), lambda i: (0, 0)),
            pl.BlockSpec(lin0_W.shape, lambda i: (0, 0)),
            pl.BlockSpec((1, lin0_b.shape[0]), lambda i: (0, 0)),
            pl.BlockSpec(lin1_W.shape, lambda i: (0, 0)),
            pl.BlockSpec((1, lin1_b.shape[0]), lambda i: (0, 0)),
        ],
        out_specs=pl.BlockSpec((rb, out_ch), lambda i: (i, 0)),
        out_shape=jax.ShapeDtypeStruct((n, out_ch), jnp.float32),
    )(a2, a_bf, x, mask8, alpha,
      fusion_W, fusion_b.reshape(1, -1),
      lin0_W, lin0_b.reshape(1, -1),
      lin1_W, lin1_b.reshape(1, -1))
    return out
